# baseline (speedup 1.0000x reference)
import functools

import jax
import jax.numpy as jnp
from jax import lax
from jax.experimental import pallas as pl
from jax.experimental.pallas import tpu as pltpu
from jax.experimental.pallas import tpu_sc as plsc

EMBED = 64
PAIR_W = 2 * EMBED
NUM_WORKERS = 32
CHUNK = 800
PAIRS = CHUNK // 2
NRAW = 4
NROW = 2
LANES = 16
GROUPS = PAIRS // LANES


def _perm(v, idx):
    return lax.gather(
        v,
        idx[:, None],
        lax.GatherDimensionNumbers(
            offset_dims=(), collapsed_slice_dims=(0,), start_index_map=(0,)
        ),
        slice_sizes=(1,),
        mode=lax.GatherScatterMode.PROMISE_IN_BOUNDS,
    )


def _sc_embed(city_flat, ptab):
    b_total = city_flat.shape[0]
    b_per_w = b_total // NUM_WORKERS
    iters = b_per_w // CHUNK
    nf = iters // NRAW
    mesh = plsc.VectorSubcoreMesh(core_axis_name="c", subcore_axis_name="s")

    @functools.partial(
        pl.kernel,
        mesh=mesh,
        out_type=jax.ShapeDtypeStruct((b_total // 2, PAIR_W), jnp.float32),
        scratch_types=[
            pltpu.VMEM((NRAW * CHUNK,), jnp.int32),
            pltpu.VMEM((NRAW * PAIRS,), jnp.int32),
            pltpu.VMEM((NROW * PAIRS, PAIR_W), jnp.float32),
            pltpu.SemaphoreType.DMA,
            pltpu.SemaphoreType.DMA,
            pltpu.SemaphoreType.DMA,
            pltpu.SemaphoreType.DMA,
            pltpu.SemaphoreType.DMA,
            pltpu.SemaphoreType.DMA,
            pltpu.SemaphoreType.DMA,
            pltpu.SemaphoreType.DMA,
        ],
    )
    def kern(city_hbm, tab_hbm, out_hbm, raw_v, qidx_v, rows_v,
             si0, si1, si2, si3, sg0, sg1, ss0, ss1):
        si = [si0, si1, si2, si3]
        sg, ss = [sg0, sg1], [ss0, ss1]
        wid = lax.axis_index("s") * 2 + lax.axis_index("c")
        w_base = wid * b_per_w
        lane = lax.iota(jnp.int32, LANES)
        even = (2 * lane) & (LANES - 1)
        odd = even + 1
        lo_half = lane < (LANES // 2)

        def rows_buf(b):
            return rows_v.at[pl.ds(b * PAIRS, PAIRS)]

        def idx_start(j, r):
            pltpu.async_copy(
                city_hbm.at[pl.ds(w_base + j * CHUNK, CHUNK)],
                raw_v.at[pl.ds(r * CHUNK, CHUNK)],
                si[r],
            )

        def idx_wait(j, r):
            pltpu.make_async_copy(
                city_hbm.at[pl.ds(w_base + j * CHUNK, CHUNK)],
                raw_v.at[pl.ds(r * CHUNK, CHUNK)],
                si[r],
            ).wait()

        desc = ((0, 104), (104, 96), (200, 104), (304, 96))

        def gather_start(r, b):
            for off, n in desc:
                pltpu.async_copy(
                    tab_hbm.at[qidx_v.at[pl.ds(r * PAIRS + off, n)]],
                    rows_buf(b).at[pl.ds(off, n)],
                    sg[b],
                )

        def gather_wait(r, b):
            for off, n in desc:
                pltpu.make_async_copy(
                    tab_hbm.at[qidx_v.at[pl.ds(r * PAIRS + off, n)]],
                    rows_buf(b).at[pl.ds(off, n)],
                    sg[b],
                ).wait()

        def out_slice(j):
            out_row = pl.multiple_of((w_base + j * CHUNK) // 2, 8)
            return out_hbm.at[pl.ds(out_row, PAIRS)]

        def store_start(j, b):
            pltpu.async_copy(rows_buf(b), out_slice(j), ss[b])

        def store_wait(j, b):
            pltpu.make_async_copy(rows_buf(b), out_slice(j), ss[b]).wait()

        def compute_qidx(r):
            for g in range(GROUPS):
                v0 = raw_v[pl.ds(r * CHUNK + 2 * LANES * g, LANES)]
                v1 = raw_v[pl.ds(r * CHUNK + 2 * LANES * g + LANES, LANES)]
                q0 = _perm(v0, even) * 5 + _perm(v0, odd)
                q1 = _perm(v1, even) * 5 + _perm(v1, odd)
                merged = jnp.where(lo_half, q0, q1)
                qidx_v[pl.ds(r * PAIRS + g * LANES, LANES)] = merged + wid * 25

        def segment(j, j2, s):
            r, b = s, s % NROW

            @pl.when(j + NRAW - 1 < iters)
            def _():
                idx_start(j + NRAW - 1, (s + NRAW - 1) % NRAW)

            idx_wait(j, r)
            compute_qidx(r)

            if s < NROW:
                pl.when(j2 >= 1)(lambda: store_wait(j - NROW, b))
            else:
                store_wait(j - NROW, b)

            gather_start(r, b)

            rp, bp = (s - 1) % NRAW, (s - 1) % NROW
            if s == 0:
                @pl.when(j2 >= 1)
                def _():
                    gather_wait(rp, bp)
                    store_start(j - 1, bp)
            else:
                gather_wait(rp, bp)
                store_start(j - 1, bp)

        for j in range(NRAW - 1):
            idx_start(j, j)

        def body(j2, carry):
            for s in range(NRAW):
                segment(NRAW * j2 + s, j2, s)
            return carry

        lax.fori_loop(0, nf, body, 0)

        last = iters - 1
        rl, bl = last % NRAW, last % NROW
        gather_wait(rl, bl)
        store_start(last, bl)
        store_wait(last - 1, (last - 1) % NROW)
        store_wait(last, bl)

    return kern(city_flat, ptab)


def kernel(city, table):
    rows, cols = city.shape
    city_flat = city.reshape(-1).astype(jnp.int32)
    ptab = jnp.concatenate(
        [jnp.repeat(table, 5, axis=0), jnp.tile(table, (5, 1))], axis=1
    )
    out = _sc_embed(city_flat, jnp.tile(ptab, (NUM_WORKERS, 1)))
    return out.reshape(rows, cols, EMBED)


# speedup vs baseline: 1.1782x; 1.1782x over previous
import functools

import jax
import jax.numpy as jnp
from jax import lax
from jax.experimental import pallas as pl
from jax.experimental.pallas import tpu as pltpu
from jax.experimental.pallas import tpu_sc as plsc

EMBED = 64
QUAD = 4
QUAD_W = QUAD * EMBED
NUM_WORKERS = 32
CHUNK = 800
QUADS = CHUNK // QUAD
QPAD = 208
RAW_W = 832
NRAW = 4
NROW = 2
LANES = 16
GROUPS = QPAD // LANES


def _perm(v, idx):
    return lax.gather(
        v,
        idx[:, None],
        lax.GatherDimensionNumbers(
            offset_dims=(), collapsed_slice_dims=(0,), start_index_map=(0,)
        ),
        slice_sizes=(1,),
        mode=lax.GatherScatterMode.PROMISE_IN_BOUNDS,
    )


def _sc_embed(city_flat, qtab):
    b_total = city_flat.shape[0]
    b_per_w = b_total // NUM_WORKERS
    iters = b_per_w // CHUNK
    nf = iters // NRAW
    mesh = plsc.VectorSubcoreMesh(core_axis_name="c", subcore_axis_name="s")

    @functools.partial(
        pl.kernel,
        mesh=mesh,
        out_type=jax.ShapeDtypeStruct((b_total // QUAD, QUAD_W), jnp.float32),
        scratch_types=[
            pltpu.VMEM((NRAW * RAW_W,), jnp.int32),
            pltpu.VMEM((NRAW * QPAD,), jnp.int32),
            pltpu.VMEM((NROW * QUADS, QUAD_W), jnp.float32),
            pltpu.SemaphoreType.DMA,
            pltpu.SemaphoreType.DMA,
            pltpu.SemaphoreType.DMA,
            pltpu.SemaphoreType.DMA,
            pltpu.SemaphoreType.DMA,
            pltpu.SemaphoreType.DMA,
            pltpu.SemaphoreType.DMA,
            pltpu.SemaphoreType.DMA,
        ],
    )
    def kern(city_hbm, tab_hbm, out_hbm, raw_v, qidx_v, rows_v,
             si0, si1, si2, si3, sg0, sg1, ss0, ss1):
        si = [si0, si1, si2, si3]
        sg, ss = [sg0, sg1], [ss0, ss1]
        wid = lax.axis_index("s") * 2 + lax.axis_index("c")
        w_base = wid * b_per_w
        lane = lax.iota(jnp.int32, LANES)
        perms = [(QUAD * lane + c) & (LANES - 1) for c in range(QUAD)]
        m0, m1, m2 = lane < 4, lane < 8, lane < 12

        def rows_buf(b):
            return rows_v.at[pl.ds(b * QUADS, QUADS)]

        def idx_start(j, r):
            pltpu.async_copy(
                city_hbm.at[pl.ds(w_base + j * CHUNK, CHUNK)],
                raw_v.at[pl.ds(r * RAW_W, CHUNK)],
                si[r],
            )

        def idx_wait(j, r):
            pltpu.make_async_copy(
                city_hbm.at[pl.ds(w_base + j * CHUNK, CHUNK)],
                raw_v.at[pl.ds(r * RAW_W, CHUNK)],
                si[r],
            ).wait()

        desc = ((0, 104), (104, 96))

        def gather_start(r, b):
            for off, n in desc:
                pltpu.async_copy(
                    tab_hbm.at[qidx_v.at[pl.ds(r * QPAD + off, n)]],
                    rows_buf(b).at[pl.ds(off, n)],
                    sg[b],
                )

        def gather_wait(r, b):
            for off, n in desc:
                pltpu.make_async_copy(
                    tab_hbm.at[qidx_v.at[pl.ds(r * QPAD + off, n)]],
                    rows_buf(b).at[pl.ds(off, n)],
                    sg[b],
                ).wait()

        def out_slice(j):
            out_row = pl.multiple_of((w_base + j * CHUNK) // QUAD, 8)
            return out_hbm.at[pl.ds(out_row, QUADS)]

        def store_start(j, b):
            pltpu.async_copy(rows_buf(b), out_slice(j), ss[b])

        def store_wait(j, b):
            pltpu.make_async_copy(rows_buf(b), out_slice(j), ss[b]).wait()

        def compute_qidx(r):
            for g in range(GROUPS):
                qs = []
                for i in range(QUAD):
                    v = raw_v[pl.ds(r * RAW_W + g * 4 * LANES + i * LANES, LANES)]
                    q = _perm(v, perms[0])
                    for c in range(1, QUAD):
                        q = q * 5 + _perm(v, perms[c])
                    qs.append(q)
                merged = jnp.where(
                    m0, qs[0], jnp.where(m1, qs[1], jnp.where(m2, qs[2], qs[3]))
                )
                qidx_v[pl.ds(r * QPAD + g * LANES, LANES)] = merged + wid * 625

        def segment(j, j2, s):
            r, b = s, s % NROW

            @pl.when(j + NRAW - 1 < iters)
            def _():
                idx_start(j + NRAW - 1, (s + NRAW - 1) % NRAW)

            idx_wait(j, r)
            compute_qidx(r)

            if s < NROW:
                pl.when(j2 >= 1)(lambda: store_wait(j - NROW, b))
            else:
                store_wait(j - NROW, b)

            gather_start(r, b)

            rp, bp = (s - 1) % NRAW, (s - 1) % NROW
            if s == 0:
                @pl.when(j2 >= 1)
                def _():
                    gather_wait(rp, bp)
                    store_start(j - 1, bp)
            else:
                gather_wait(rp, bp)
                store_start(j - 1, bp)

        for j in range(NRAW - 1):
            idx_start(j, j)

        def body(j2, carry):
            for s in range(NRAW):
                segment(NRAW * j2 + s, j2, s)
            return carry

        lax.fori_loop(0, nf, body, 0)

        last = iters - 1
        rl, bl = last % NRAW, last % NROW
        gather_wait(rl, bl)
        store_start(last, bl)
        store_wait(last - 1, (last - 1) % NROW)
        store_wait(last, bl)

    return kern(city_flat, qtab)


def kernel(city, table):
    rows, cols = city.shape
    city_flat = city.reshape(-1).astype(jnp.int32)
    qtab = jnp.concatenate(
        [
            jnp.repeat(table, 125, axis=0),
            jnp.tile(jnp.repeat(table, 25, axis=0), (5, 1)),
            jnp.tile(jnp.repeat(table, 5, axis=0), (25, 1)),
            jnp.tile(table, (125, 1)),
        ],
        axis=1,
    )
    out = _sc_embed(city_flat, jnp.tile(qtab, (NUM_WORKERS, 1)))
    return out.reshape(rows, cols, EMBED)
